# baseline (speedup 1.0000x reference)
import functools

import jax
import jax.numpy as jnp
from jax import lax
from jax.experimental import pallas as pl
from jax.experimental.pallas import tpu as pltpu
from jax.experimental.pallas import tpu_sc as plsc

B, L, E = 4096, 200, 32
LE = L * E
NC, NS = 2, 16
NW = NC * NS
RPW = B // NW
SPLIT = 128
REST = L - SPLIT
NBUF = 4
BB = 128


def _sc_gather(inputs, table):
  mesh = plsc.VectorSubcoreMesh(core_axis_name="c", subcore_axis_name="s")

  @functools.partial(
      pl.kernel,
      out_type=jax.ShapeDtypeStruct((B * L, E), jnp.float32),
      mesh=mesh,
      compiler_params=pltpu.CompilerParams(
          use_tc_tiling_on_sc=False, needs_layout_passes=False),
      scratch_types=[
          pltpu.VMEM((RPW, L), jnp.int32),
          pltpu.VMEM((NBUF, L, E), jnp.float32),
          pltpu.SemaphoreType.DMA((NBUF,)),
          pltpu.SemaphoreType.DMA((NBUF,)),
      ],
  )
  def k(inputs_hbm, table_hbm, emb_hbm, idx_v, bufs, gsem, osem):
    wid = lax.axis_index("s") * NC + lax.axis_index("c")
    base = wid * RPW

    pltpu.sync_copy(inputs_hbm.at[pl.ds(base, RPW)], idx_v)

    def gather_row(r, b, start):
      cp0 = pltpu.make_async_copy(
          table_hbm.at[idx_v.at[r, pl.ds(0, SPLIT)]],
          bufs.at[b, pl.ds(0, SPLIT)], gsem.at[b])
      cp1 = pltpu.make_async_copy(
          table_hbm.at[idx_v.at[r, pl.ds(SPLIT, REST)]],
          bufs.at[b, pl.ds(SPLIT, REST)], gsem.at[b])
      if start:
        cp0.start()
        cp1.start()
      else:
        cp0.wait()
        cp1.wait()

    def out_row(r, b, start):
      cp = pltpu.make_async_copy(bufs.at[b],
                                 emb_hbm.at[pl.ds((base + r) * L, L)],
                                 osem.at[b])
      if start:
        cp.start()
      else:
        cp.wait()

    for x in range(NBUF - 1):
      gather_row(x, x, True)

    def group(g, carry):
      for b in range(NBUF):
        r = g * NBUF + b
        bprev = (b - 1) % NBUF
        gather_row(r, b, False)
        out_row(r, b, True)

        @pl.when(r >= 1)
        def _():
          out_row(r - 1, bprev, False)

        @pl.when(r + NBUF - 1 < RPW)
        def _():
          gather_row(r + NBUF - 1, bprev, True)

      return carry

    lax.fori_loop(0, RPW // NBUF, group, 0)
    out_row(RPW - 1, (RPW - 1) % NBUF, False)

  return k(inputs, table)


VC = 4096


def _tc_table_rm(tableT):
  def body(t_ref, o_ref):
    o_ref[...] = jnp.transpose(t_ref[...])

  vocab = tableT.shape[1]
  return pl.pallas_call(
      body,
      grid=((vocab + VC - 1) // VC,),
      in_specs=[pl.BlockSpec((E, VC), lambda i: (0, i))],
      out_specs=pl.BlockSpec((VC, E), lambda i: (i, 0)),
      out_shape=jax.ShapeDtypeStruct((vocab, E), jnp.float32),
  )(tableT)


def _tc_finish(embw, fc_W, fc_b, cls_W, cls_b):
  def body(x_ref, w1_ref, b1_ref, w2_ref, b2_ref, t_ref, p_ref, fc_ref,
           lg_ref):
    x = x_ref[...]
    xt = jnp.transpose(x)
    t_ref[...] = xt
    xt3 = xt.reshape(L, E, BB)
    poolT = jnp.sum(xt3, axis=0) * jnp.float32(1.0 / L)
    p_ref[...] = poolT
    w1t = jnp.transpose(w1_ref[...])
    fc = jnp.maximum(
        jnp.dot(w1t, poolT, preferred_element_type=jnp.float32)
        + b1_ref[...], 0.0)
    fc_ref[...] = fc
    w2t = jnp.transpose(w2_ref[...])
    lg_ref[...] = jnp.dot(
        w2t, fc, preferred_element_type=jnp.float32) + b2_ref[...]

  nblk = B // BB
  return pl.pallas_call(
      body,
      grid=(nblk,),
      in_specs=[
          pl.BlockSpec((BB, LE), lambda i: (i, 0)),
          pl.BlockSpec((E, 64), lambda i: (0, 0)),
          pl.BlockSpec((64, 1), lambda i: (0, 0)),
          pl.BlockSpec((64, 2), lambda i: (0, 0)),
          pl.BlockSpec((2, 1), lambda i: (0, 0)),
      ],
      out_specs=(
          pl.BlockSpec((LE, BB), lambda i: (0, i)),
          pl.BlockSpec((E, BB), lambda i: (0, i)),
          pl.BlockSpec((64, BB), lambda i: (0, i)),
          pl.BlockSpec((2, BB), lambda i: (0, i)),
      ),
      out_shape=(
          jax.ShapeDtypeStruct((LE, B), jnp.float32),
          jax.ShapeDtypeStruct((E, B), jnp.float32),
          jax.ShapeDtypeStruct((64, B), jnp.float32),
          jax.ShapeDtypeStruct((2, B), jnp.float32),
      ),
  )(embw, fc_W, fc_b.reshape(64, 1), cls_W, cls_b.reshape(2, 1))


def kernel(inputs, table, fc_W, fc_b, cls_W, cls_b):
  table_rm = _tc_table_rm(jnp.transpose(table))
  myout = _sc_gather(inputs, table_rm)
  embw = myout.reshape(B, LE)
  embT, poolT, fcT, logitsT = _tc_finish(embw, fc_W, fc_b, cls_W, cls_b)
  emb = jnp.transpose(embT.reshape(L, E, B), (2, 0, 1))
  return (jnp.transpose(logitsT), emb, jnp.transpose(poolT),
          jnp.transpose(fcT))


# speedup vs baseline: 1.2081x; 1.2081x over previous
import functools

import jax
import jax.numpy as jnp
from jax import lax
from jax.experimental import pallas as pl
from jax.experimental.pallas import tpu as pltpu
from jax.experimental.pallas import tpu_sc as plsc

B, L, E = 4096, 200, 32
LE = L * E
NC, NS = 2, 16
NW = NC * NS
RPW = B // NW
SPLIT = 128
REST = L - SPLIT
NBUF = 4
BB = 128


def _sc_gather(inputs, table):
  mesh = plsc.VectorSubcoreMesh(core_axis_name="c", subcore_axis_name="s")

  @functools.partial(
      pl.kernel,
      out_type=jax.ShapeDtypeStruct((B * L, E), jnp.float32),
      mesh=mesh,
      compiler_params=pltpu.CompilerParams(
          use_tc_tiling_on_sc=False, needs_layout_passes=False),
      scratch_types=[
          pltpu.VMEM((RPW, L), jnp.int32),
          pltpu.VMEM((NBUF, L, E), jnp.float32),
          pltpu.SemaphoreType.DMA((NBUF,)),
          pltpu.SemaphoreType.DMA((NBUF,)),
      ],
  )
  def k(inputs_hbm, table_hbm, emb_hbm, idx_v, bufs, gsem, osem):
    wid = lax.axis_index("s") * NC + lax.axis_index("c")
    base = wid * RPW

    pltpu.sync_copy(inputs_hbm.at[pl.ds(base, RPW)], idx_v)

    def gather_row(r, b, start):
      cp0 = pltpu.make_async_copy(
          table_hbm.at[idx_v.at[r, pl.ds(0, SPLIT)]],
          bufs.at[b, pl.ds(0, SPLIT)], gsem.at[b])
      cp1 = pltpu.make_async_copy(
          table_hbm.at[idx_v.at[r, pl.ds(SPLIT, REST)]],
          bufs.at[b, pl.ds(SPLIT, REST)], gsem.at[b])
      if start:
        cp0.start()
        cp1.start()
      else:
        cp0.wait()
        cp1.wait()

    def out_row(r, b, start):
      cp = pltpu.make_async_copy(bufs.at[b],
                                 emb_hbm.at[pl.ds((base + r) * L, L)],
                                 osem.at[b])
      if start:
        cp.start()
      else:
        cp.wait()

    for x in range(NBUF - 1):
      gather_row(x, x, True)

    def group(g, carry):
      for b in range(NBUF):
        r = g * NBUF + b
        bprev = (b - 1) % NBUF
        gather_row(r, b, False)
        out_row(r, b, True)

        @pl.when(r >= 1)
        def _():
          out_row(r - 1, bprev, False)

        @pl.when(r + NBUF - 1 < RPW)
        def _():
          gather_row(r + NBUF - 1, bprev, True)

      return carry

    lax.fori_loop(0, RPW // NBUF, group, 0)
    out_row(RPW - 1, (RPW - 1) % NBUF, False)

  return k(inputs, table)


def _tc_finish(embw, fc_W, fc_b, cls_W, cls_b):
  def body(x_ref, w1_ref, b1_ref, w2_ref, b2_ref, t_ref, p_ref, fc_ref,
           lg_ref):
    x = x_ref[...]
    xt = jnp.transpose(x)
    t_ref[...] = xt
    xt3 = xt.reshape(L, E, BB)
    poolT = jnp.sum(xt3, axis=0) * jnp.float32(1.0 / L)
    p_ref[...] = poolT
    w1t = jnp.transpose(w1_ref[...])
    fc = jnp.maximum(
        jnp.dot(w1t, poolT, preferred_element_type=jnp.float32)
        + b1_ref[...], 0.0)
    fc_ref[...] = fc
    w2t = jnp.transpose(w2_ref[...])
    lg_ref[...] = jnp.dot(
        w2t, fc, preferred_element_type=jnp.float32) + b2_ref[...]

  nblk = B // BB
  return pl.pallas_call(
      body,
      grid=(nblk,),
      in_specs=[
          pl.BlockSpec((BB, LE), lambda i: (i, 0)),
          pl.BlockSpec((E, 64), lambda i: (0, 0)),
          pl.BlockSpec((64, 1), lambda i: (0, 0)),
          pl.BlockSpec((64, 2), lambda i: (0, 0)),
          pl.BlockSpec((2, 1), lambda i: (0, 0)),
      ],
      out_specs=(
          pl.BlockSpec((LE, BB), lambda i: (0, i)),
          pl.BlockSpec((E, BB), lambda i: (0, i)),
          pl.BlockSpec((64, BB), lambda i: (0, i)),
          pl.BlockSpec((2, BB), lambda i: (0, i)),
      ),
      out_shape=(
          jax.ShapeDtypeStruct((LE, B), jnp.float32),
          jax.ShapeDtypeStruct((E, B), jnp.float32),
          jax.ShapeDtypeStruct((64, B), jnp.float32),
          jax.ShapeDtypeStruct((2, B), jnp.float32),
      ),
  )(embw, fc_W, fc_b.reshape(64, 1), cls_W, cls_b.reshape(2, 1))


def kernel(inputs, table, fc_W, fc_b, cls_W, cls_b):
  myout = _sc_gather(inputs, table)
  embw = myout.reshape(B, LE)
  embT, poolT, fcT, logitsT = _tc_finish(embw, fc_W, fc_b, cls_W, cls_b)
  emb = jnp.transpose(embT.reshape(L, E, B), (2, 0, 1))
  return (jnp.transpose(logitsT), emb, jnp.transpose(poolT),
          jnp.transpose(fcT))
